# baseline (speedup 1.0000x reference)
import math

import jax
import jax.numpy as jnp
from jax.experimental import pallas as pl
from jax.experimental.pallas import tpu as pltpu

QDIM = 256
MID_D = 256
N_MAIN = 10000
N_OTHER = 1024
B_FIX = 4096
FCHUNK = 1024
NF = B_FIX // FCHUNK
BM = 1000
NA = N_MAIN // BM
SCALE = 1.0 / math.sqrt(MID_D)


def _bf(x):
    return x.astype(jnp.bfloat16)


def _dot(a, b, dims):
    return jax.lax.dot_general(_bf(a), _bf(b), (dims, ((), ())),
                               preferred_element_type=jnp.float32)


H = N_OTHER // 2


def _fused_kernel(fix_ref, other_ref, wk_ref, bk_ref, main_ref, wq_ref,
                  bq_ref, out_ref, g00_ref, g01_ref, g11_ref,
                  om_ref, k_ref):
    i = pl.program_id(0)

    @pl.when(i == 0)
    def _gram_first():
        fl = _bf(fix_ref[:, :H])
        fr = _bf(fix_ref[:, H:])
        g00_ref[...] = _dot(fl, fl, ((0,), (0,)))
        g01_ref[...] = _dot(fl, fr, ((0,), (0,)))
        g11_ref[...] = _dot(fr, fr, ((0,), (0,)))

    @pl.when(jnp.logical_and(i > 0, i < NF))
    def _gram_acc():
        fl = _bf(fix_ref[:, :H])
        fr = _bf(fix_ref[:, H:])
        g00_ref[...] += _dot(fl, fl, ((0,), (0,)))
        g01_ref[...] += _dot(fl, fr, ((0,), (0,)))
        g11_ref[...] += _dot(fr, fr, ((0,), (0,)))

    @pl.when(i == NF - 1)
    def _preamble_finish():
        f00 = jnp.sqrt(g00_ref[...])
        f01 = jnp.sqrt(g01_ref[...])
        f11 = jnp.sqrt(g11_ref[...])
        colsum_l = jnp.sum(f00, axis=0) + jnp.sum(f01, axis=1)
        colsum_r = jnp.sum(f01, axis=0) + jnp.sum(f11, axis=0)
        other = other_ref[...]
        so_l = other[:H, :] / colsum_l[:, None]
        so_r = other[H:, :] / colsum_r[:, None]
        om_ref[:H, :MID_D] = _bf(_dot(f00, so_l, ((1,), (0,)))
                                 + _dot(f01, so_r, ((1,), (0,))))
        om_ref[H:, :MID_D] = _bf(_dot(f01, so_l, ((0,), (0,)))
                                 + _dot(f11, so_r, ((1,), (0,))))
        om_ref[:, MID_D:] = jnp.ones((N_OTHER, 128), dtype=jnp.bfloat16)
        k = _dot(other, wk_ref[...], ((1,), (1,))) + bk_ref[...]
        k_ref[...] = _bf(k * SCALE)

    @pl.when(i >= NF)
    def _attention():
        q = _dot(main_ref[...], wq_ref[...], ((1,), (1,))) + bq_ref[...]
        a = _dot(q, k_ref[...], ((1,), (1,)))
        p = jnp.exp(a).astype(jnp.bfloat16)
        o_aug = _dot(p, om_ref[...], ((1,), (0,)))
        out_ref[...] = o_aug[:, :MID_D] * (1.0 / o_aug[:, MID_D:MID_D + 1])


def kernel(main_feat, other_feat, fix_feat, Wq, bq, Wk, bk):
    bq2 = bq.reshape(1, MID_D)
    bk2 = bk.reshape(1, MID_D)

    O = pl.pallas_call(
        _fused_kernel,
        grid=(NF + NA,),
        in_specs=[
            pl.BlockSpec((FCHUNK, N_OTHER), lambda i: (jnp.minimum(i, NF - 1), 0)),
            pl.BlockSpec((N_OTHER, MID_D), lambda i: (0, 0)),
            pl.BlockSpec((MID_D, MID_D), lambda i: (0, 0)),
            pl.BlockSpec((1, MID_D), lambda i: (0, 0)),
            pl.BlockSpec((BM, QDIM), lambda i: (jnp.maximum(i - NF, 0), 0)),
            pl.BlockSpec((MID_D, QDIM), lambda i: (0, 0)),
            pl.BlockSpec((1, MID_D), lambda i: (0, 0)),
        ],
        out_specs=pl.BlockSpec((BM, MID_D), lambda i: (jnp.maximum(i - NF, 0), 0)),
        out_shape=jax.ShapeDtypeStruct((N_MAIN, MID_D), jnp.float32),
        scratch_shapes=[
            pltpu.VMEM((H, H), jnp.float32),
            pltpu.VMEM((H, H), jnp.float32),
            pltpu.VMEM((H, H), jnp.float32),
            pltpu.VMEM((N_OTHER, MID_D + 128), jnp.bfloat16),
            pltpu.VMEM((N_OTHER, MID_D), jnp.bfloat16),
        ],
        compiler_params=pltpu.CompilerParams(
            dimension_semantics=("arbitrary",),
        ),
    )(fix_feat, other_feat, Wk, bk2, main_feat, Wq, bq2)
    return O


# speedup vs baseline: 1.0486x; 1.0486x over previous
import math

import jax
import jax.numpy as jnp
from jax.experimental import pallas as pl
from jax.experimental.pallas import tpu as pltpu

QDIM = 256
MID_D = 256
N_MAIN = 10000
N_OTHER = 1024
B_FIX = 4096
FCHUNK = 1024
NF = B_FIX // FCHUNK
BM = 2000
NA = N_MAIN // BM
SCALE = 1.0 / math.sqrt(MID_D)
LOG2E = math.log2(math.e)


def _bf(x):
    return x.astype(jnp.bfloat16)


def _dot(a, b, dims):
    return jax.lax.dot_general(_bf(a), _bf(b), (dims, ((), ())),
                               preferred_element_type=jnp.float32)


H = N_OTHER // 2


def _fused_kernel(fix_ref, other_ref, wk_ref, bk_ref, main_ref, wq_ref,
                  bq_ref, out_ref, g00_ref, g01_ref, g11_ref,
                  om_ref, k_ref):
    i = pl.program_id(0)

    @pl.when(i == 0)
    def _gram_first():
        fl = _bf(fix_ref[:, :H])
        fr = _bf(fix_ref[:, H:])
        g00_ref[...] = _dot(fl, fl, ((0,), (0,)))
        g01_ref[...] = _dot(fl, fr, ((0,), (0,)))
        g11_ref[...] = _dot(fr, fr, ((0,), (0,)))

    @pl.when(jnp.logical_and(i > 0, i < NF))
    def _gram_acc():
        fl = _bf(fix_ref[:, :H])
        fr = _bf(fix_ref[:, H:])
        g00_ref[...] += _dot(fl, fl, ((0,), (0,)))
        g01_ref[...] += _dot(fl, fr, ((0,), (0,)))
        g11_ref[...] += _dot(fr, fr, ((0,), (0,)))

    @pl.when(i == NF - 1)
    def _preamble_finish():
        f00 = jnp.sqrt(g00_ref[...])
        f01 = jnp.sqrt(g01_ref[...])
        f11 = jnp.sqrt(g11_ref[...])
        colsum_l = jnp.sum(f00, axis=0) + jnp.sum(f01, axis=1)
        colsum_r = jnp.sum(f01, axis=0) + jnp.sum(f11, axis=0)
        other = other_ref[...]
        so_l = other[:H, :] / colsum_l[:, None]
        so_r = other[H:, :] / colsum_r[:, None]
        om_ref[:H, :MID_D] = _bf(_dot(f00, so_l, ((1,), (0,)))
                                 + _dot(f01, so_r, ((1,), (0,))))
        om_ref[H:, :MID_D] = _bf(_dot(f01, so_l, ((0,), (0,)))
                                 + _dot(f11, so_r, ((1,), (0,))))
        om_ref[:, MID_D:] = jnp.ones((N_OTHER, 128), dtype=jnp.bfloat16)
        k = _dot(other, wk_ref[...], ((1,), (1,))) + bk_ref[...]
        k_ref[...] = _bf(k * (SCALE * LOG2E))

    @pl.when(i >= NF)
    def _attention():
        q = _dot(main_ref[...], wq_ref[...], ((1,), (1,))) + bq_ref[...]
        a = _dot(q, k_ref[...], ((1,), (1,)))
        p = jnp.exp2(a).astype(jnp.bfloat16)
        o_aug = _dot(p, om_ref[...], ((1,), (0,)))
        out_ref[...] = o_aug[:, :MID_D] * (1.0 / o_aug[:, MID_D:MID_D + 1])


def kernel(main_feat, other_feat, fix_feat, Wq, bq, Wk, bk):
    bq2 = bq.reshape(1, MID_D)
    bk2 = bk.reshape(1, MID_D)

    O = pl.pallas_call(
        _fused_kernel,
        grid=(NF + NA,),
        in_specs=[
            pl.BlockSpec((FCHUNK, N_OTHER), lambda i: (jnp.minimum(i, NF - 1), 0)),
            pl.BlockSpec((N_OTHER, MID_D), lambda i: (0, 0)),
            pl.BlockSpec((MID_D, MID_D), lambda i: (0, 0)),
            pl.BlockSpec((1, MID_D), lambda i: (0, 0)),
            pl.BlockSpec((BM, QDIM), lambda i: (jnp.maximum(i - NF, 0), 0)),
            pl.BlockSpec((MID_D, QDIM), lambda i: (0, 0)),
            pl.BlockSpec((1, MID_D), lambda i: (0, 0)),
        ],
        out_specs=pl.BlockSpec((BM, MID_D), lambda i: (jnp.maximum(i - NF, 0), 0)),
        out_shape=jax.ShapeDtypeStruct((N_MAIN, MID_D), jnp.float32),
        scratch_shapes=[
            pltpu.VMEM((H, H), jnp.float32),
            pltpu.VMEM((H, H), jnp.float32),
            pltpu.VMEM((H, H), jnp.float32),
            pltpu.VMEM((N_OTHER, MID_D + 128), jnp.bfloat16),
            pltpu.VMEM((N_OTHER, MID_D), jnp.bfloat16),
        ],
        compiler_params=pltpu.CompilerParams(
            dimension_semantics=("arbitrary",),
        ),
    )(fix_feat, other_feat, Wk, bk2, main_feat, Wq, bq2)
    return O


# speedup vs baseline: 1.1010x; 1.0499x over previous
import math

import jax
import jax.numpy as jnp
from jax.experimental import pallas as pl
from jax.experimental.pallas import tpu as pltpu

QDIM = 256
MID_D = 256
N_MAIN = 10000
N_OTHER = 1024
B_FIX = 4096
FCHUNK = 1024
NF = B_FIX // FCHUNK
BM = 2000
NA = N_MAIN // BM
SCALE = 1.0 / math.sqrt(MID_D)
LOG2E = math.log2(math.e)


def _bf(x):
    return x.astype(jnp.bfloat16)


def _dot(a, b, dims):
    return jax.lax.dot_general(_bf(a), _bf(b), (dims, ((), ())),
                               preferred_element_type=jnp.float32)


H = N_OTHER // 2


def _fused_kernel(fix_ref, other_ref, wk_ref, bk_ref, main_ref, wq_ref,
                  bq_ref, out_ref, g00_ref, g01_ref, g11_ref,
                  om_ref, k_ref, brow_ref):
    i = pl.program_id(0)

    @pl.when(i == 0)
    def _gram_first():
        fl = _bf(fix_ref[:, :H])
        fr = _bf(fix_ref[:, H:])
        g00_ref[...] = _dot(fl, fl, ((0,), (0,)))
        g01_ref[...] = _dot(fl, fr, ((0,), (0,)))
        g11_ref[...] = _dot(fr, fr, ((0,), (0,)))

    @pl.when(jnp.logical_and(i > 0, i < NF))
    def _gram_acc():
        fl = _bf(fix_ref[:, :H])
        fr = _bf(fix_ref[:, H:])
        g00_ref[...] += _dot(fl, fl, ((0,), (0,)))
        g01_ref[...] += _dot(fl, fr, ((0,), (0,)))
        g11_ref[...] += _dot(fr, fr, ((0,), (0,)))

    @pl.when(i == NF - 1)
    def _preamble_finish():
        f00 = jnp.sqrt(g00_ref[...])
        f01 = jnp.sqrt(g01_ref[...])
        f11 = jnp.sqrt(g11_ref[...])
        colsum_l = jnp.sum(f00, axis=0) + jnp.sum(f01, axis=1)
        colsum_r = jnp.sum(f01, axis=0) + jnp.sum(f11, axis=0)
        other = other_ref[...]
        so_l = other[:H, :] / colsum_l[:, None]
        so_r = other[H:, :] / colsum_r[:, None]
        om_ref[:H, :MID_D] = _bf(_dot(f00, so_l, ((1,), (0,)))
                                 + _dot(f01, so_r, ((1,), (0,))))
        om_ref[H:, :MID_D] = _bf(_dot(f01, so_l, ((0,), (0,)))
                                 + _dot(f11, so_r, ((1,), (0,))))
        om_ref[:, MID_D:] = jnp.ones((N_OTHER, 128), dtype=jnp.bfloat16)
        k = _dot(other, wk_ref[...], ((1,), (1,))) + bk_ref[...]
        k_ref[...] = _bf(_dot(k, wq_ref[...], ((1,), (0,))) * (SCALE * LOG2E))
        brow_ref[...] = _dot(bq_ref[...], k, ((1,), (1,))) * (SCALE * LOG2E)

    @pl.when(i >= NF)
    def _attention():
        a = _dot(main_ref[...], k_ref[...], ((1,), (1,))) + brow_ref[...]
        p = jnp.exp2(a).astype(jnp.bfloat16)
        o_aug = _dot(p, om_ref[...], ((1,), (0,)))
        out_ref[...] = o_aug[:, :MID_D] * (1.0 / o_aug[:, MID_D:MID_D + 1])


def kernel(main_feat, other_feat, fix_feat, Wq, bq, Wk, bk):
    bq2 = bq.reshape(1, MID_D)
    bk2 = bk.reshape(1, MID_D)

    O = pl.pallas_call(
        _fused_kernel,
        grid=(NF + NA,),
        in_specs=[
            pl.BlockSpec((FCHUNK, N_OTHER), lambda i: (jnp.minimum(i, NF - 1), 0)),
            pl.BlockSpec((N_OTHER, MID_D), lambda i: (0, 0)),
            pl.BlockSpec((MID_D, MID_D), lambda i: (0, 0)),
            pl.BlockSpec((1, MID_D), lambda i: (0, 0)),
            pl.BlockSpec((BM, QDIM), lambda i: (jnp.maximum(i - NF, 0), 0)),
            pl.BlockSpec((MID_D, QDIM), lambda i: (0, 0)),
            pl.BlockSpec((1, MID_D), lambda i: (0, 0)),
        ],
        out_specs=pl.BlockSpec((BM, MID_D), lambda i: (jnp.maximum(i - NF, 0), 0)),
        out_shape=jax.ShapeDtypeStruct((N_MAIN, MID_D), jnp.float32),
        scratch_shapes=[
            pltpu.VMEM((H, H), jnp.float32),
            pltpu.VMEM((H, H), jnp.float32),
            pltpu.VMEM((H, H), jnp.float32),
            pltpu.VMEM((N_OTHER, MID_D + 128), jnp.bfloat16),
            pltpu.VMEM((N_OTHER, MID_D), jnp.bfloat16),
            pltpu.VMEM((1, N_OTHER), jnp.float32),
        ],
        compiler_params=pltpu.CompilerParams(
            dimension_semantics=("arbitrary",),
        ),
    )(fix_feat, other_feat, Wk, bk2, main_feat, Wq, bq2)
    return O
